# Initial kernel scaffold; baseline (speedup 1.0000x reference)
import jax
import jax.numpy as jnp
from jax.experimental import pallas as pl


def kernel(word, context, negative, W_word, W_context):
    raise NotImplementedError("write your pallas kernel here")


# speedup vs baseline: 5.3808x; 5.3808x over previous
import functools

import jax
import jax.numpy as jnp
from jax import lax
from jax.experimental import pallas as pl
from jax.experimental.pallas import tpu as pltpu
from jax.experimental.pallas import tpu_sc as plsc

VOCAB = 1000000
EMBED_DIM = 64
BATCH = 16384
N_NEG = 20

NC = 2
NS = 16
NW = NC * NS
B_PER_W = BATCH // NW
BC = 32
N_CHUNKS = B_PER_W // BC
NEG_PER_CHUNK = BC * N_NEG
N_GATHERS = NEG_PER_CHUNK // 128


def _sc_partials(word_r, ctx_r, neg_r, W_word, W_context):
    mesh = plsc.VectorSubcoreMesh(
        core_axis_name="c", subcore_axis_name="s", num_cores=NC,
        num_subcores=NS)

    @functools.partial(
        pl.kernel,
        out_type=[
            jax.ShapeDtypeStruct((BATCH, 16), jnp.float32),
            jax.ShapeDtypeStruct((BATCH, 16), jnp.float32),
        ],
        mesh=mesh,
        compiler_params=pltpu.CompilerParams(use_tc_tiling_on_sc=False),
        scratch_types=[
            pltpu.VMEM((N_CHUNKS, BC), jnp.int32),
            pltpu.VMEM((N_CHUNKS, BC), jnp.int32),
            pltpu.VMEM((N_CHUNKS, N_GATHERS, 128), jnp.int32),
            pltpu.VMEM((BC, EMBED_DIM), jnp.float32),
            pltpu.VMEM((BC, EMBED_DIM), jnp.float32),
            pltpu.VMEM((BC, EMBED_DIM), jnp.float32),
            pltpu.VMEM((BC, EMBED_DIM), jnp.float32),
            pltpu.VMEM((NEG_PER_CHUNK, EMBED_DIM), jnp.float32),
            pltpu.VMEM((NEG_PER_CHUNK, EMBED_DIM), jnp.float32),
            pltpu.VMEM((BC, 16), jnp.float32),
            pltpu.VMEM((BC, 16), jnp.float32),
            pltpu.SemaphoreType.DMA,
            pltpu.SemaphoreType.DMA,
        ],
    )
    def k(word_hbm, ctx_hbm, neg_hbm, ww_hbm, wc_hbm,
          pos_hbm, neg_out_hbm,
          widx, cidx, nidx, wA, wB, cA, cB, nA, nB, pbuf, nbuf,
          semA, semB):
        wid = lax.axis_index("s") * NC + lax.axis_index("c")
        base = wid * B_PER_W

        pltpu.sync_copy(word_hbm.at[wid], widx)
        pltpu.sync_copy(ctx_hbm.at[wid], cidx)
        pltpu.sync_copy(neg_hbm.at[wid], nidx)

        def fire(ck, w_buf, c_buf, n_buf, sem):
            pltpu.async_copy(ww_hbm.at[widx.at[ck]], w_buf, sem)
            pltpu.async_copy(wc_hbm.at[cidx.at[ck]], c_buf, sem)
            for j in range(N_GATHERS):
                pltpu.async_copy(wc_hbm.at[nidx.at[ck, j]],
                                 n_buf.at[pl.ds(j * 128, 128)], sem)

        def drain(w_buf, c_buf, n_buf, sem):
            pltpu.make_async_copy(ww_hbm.at[pl.ds(0, BC)], w_buf, sem).wait()
            pltpu.make_async_copy(wc_hbm.at[pl.ds(0, BC)], c_buf, sem).wait()
            pltpu.make_async_copy(
                wc_hbm.at[pl.ds(0, NEG_PER_CHUNK)], n_buf, sem).wait()

        def compute(ck, w_buf, c_buf, n_buf):
            def body_b(b, carry):
                w0 = w_buf[b, pl.ds(0, 16)]
                w1 = w_buf[b, pl.ds(16, 16)]
                w2 = w_buf[b, pl.ds(32, 16)]
                w3 = w_buf[b, pl.ds(48, 16)]
                pos = (w0 * c_buf[b, pl.ds(0, 16)]
                       + w1 * c_buf[b, pl.ds(16, 16)]
                       + w2 * c_buf[b, pl.ds(32, 16)]
                       + w3 * c_buf[b, pl.ds(48, 16)])

                def body_n(n, acc):
                    r = n * BC + b
                    return (acc
                            + w0 * n_buf[r, pl.ds(0, 16)]
                            + w1 * n_buf[r, pl.ds(16, 16)]
                            + w2 * n_buf[r, pl.ds(32, 16)]
                            + w3 * n_buf[r, pl.ds(48, 16)])

                neg = lax.fori_loop(0, N_NEG, body_n,
                                    jnp.zeros((16,), jnp.float32))
                pbuf[b, :] = pos
                nbuf[b, :] = neg
                return carry

            lax.fori_loop(0, BC, body_b, 0)
            off = base + ck * BC
            pltpu.sync_copy(pbuf, pos_hbm.at[pl.ds(off, BC)])
            pltpu.sync_copy(nbuf, neg_out_hbm.at[pl.ds(off, BC)])

        fire(0, wA, cA, nA, semA)
        fire(1, wB, cB, nB, semB)

        def loop_body(i, carry):
            ck = 2 * i
            drain(wA, cA, nA, semA)
            compute(ck, wA, cA, nA)
            fire(ck + 2, wA, cA, nA, semA)
            drain(wB, cB, nB, semB)
            compute(ck + 1, wB, cB, nB)
            fire(ck + 3, wB, cB, nB, semB)
            return carry

        lax.fori_loop(0, N_CHUNKS // 2 - 1, loop_body, 0)
        drain(wA, cA, nA, semA)
        compute(N_CHUNKS - 2, wA, cA, nA)
        drain(wB, cB, nB, semB)
        compute(N_CHUNKS - 1, wB, cB, nB)

    return k(word_r, ctx_r, neg_r, W_word, W_context)


def _tc_loss(pos2, neg2):
    def body(p_ref, n_ref, o_ref):
        p = p_ref[...]
        n = n_ref[...]
        j = lax.broadcasted_iota(jnp.int32, (128, 8), 0)
        k = lax.broadcasted_iota(jnp.int32, (128, 8), 1)
        m = (j // 16 == k).astype(jnp.float32)
        sp = jnp.dot(p, m, preferred_element_type=jnp.float32)
        sn = jnp.dot(n, m, preferred_element_type=jnp.float32)
        l = jax.nn.log_sigmoid(sp) + jax.nn.log_sigmoid(-sn)
        o_ref[...] = (-jnp.sum(l) / BATCH).reshape(1, 1)

    return pl.pallas_call(
        body,
        out_shape=jax.ShapeDtypeStruct((1, 1), jnp.float32),
    )(pos2, neg2)


def kernel(word, context, negative, W_word, W_context):
    word_r = word.astype(jnp.int32).reshape(NW, N_CHUNKS, BC)
    ctx_r = context.astype(jnp.int32).reshape(NW, N_CHUNKS, BC)
    neg_r = (negative.astype(jnp.int32)
             .reshape(NW * N_CHUNKS, BC, N_NEG)
             .transpose(0, 2, 1)
             .reshape(NW, N_CHUNKS, N_GATHERS, 128))
    pos_part, neg_part = _sc_partials(word_r, ctx_r, neg_r, W_word, W_context)
    out = _tc_loss(pos_part.reshape(BATCH // 8, 128),
                   neg_part.reshape(BATCH // 8, 128))
    return out.reshape(())
